# baseline (speedup 1.0000x reference)
import jax
import jax.numpy as jnp
from jax.experimental import pallas as pl
from jax.experimental.pallas import tpu as pltpu

NE = 4
B = 256
NOBJ = 16
N = 17
AG = 8
DYN = 12
STAT = 4
EMB = 32
HID = 64
ACT = 8

BB = 64


def _gnn_kernel(agent_ref, dyn_ref, stat_ref, act_ref,
                W_ea_ref, b_ea_ref, W_eo_ref, b_eo_ref,
                W_e1_ref, b_e1_ref, g_e_ref, be_e_ref, W_e2_ref, b_e2_ref,
                W_n1_ref, b_n1_ref, g_n_ref, be_n_ref, W_n2_ref, b_n2_ref,
                W_oa_ref, b_oa_ref, W_od_ref, b_od_ref,
                agent_out_ref, obj_out_ref):
    f32 = jnp.float32

    def mm(x, w):
        return jnp.dot(x, w, preferred_element_type=f32,
                       precision=jax.lax.Precision.HIGHEST)

    def ln_relu(h, g, bb):
        m = jnp.mean(h, axis=-1, keepdims=True)
        v = jnp.mean(jnp.square(h - m), axis=-1, keepdims=True)
        return jnp.maximum((h - m) * jax.lax.rsqrt(v + 1e-5) * g + bb, 0.0)

    ag = agent_ref[0]
    act = act_ref[0]
    obj_in = jnp.concatenate([dyn_ref[0], stat_ref[0]], axis=-1)

    agent_emb = mm(ag, W_ea_ref[0]) + b_ea_ref[0]
    obj_emb = mm(obj_in.reshape(BB * NOBJ, DYN + STAT), W_eo_ref[0]) + b_eo_ref[0]
    nf = jnp.concatenate([agent_emb[:, None, :],
                          obj_emb.reshape(BB, NOBJ, EMB)], axis=1)
    nf2 = nf.reshape(BB * N, EMB)

    W_e1 = W_e1_ref[0]
    P = mm(nf2, W_e1[0:EMB]).reshape(BB, N, HID)
    Q = mm(nf2, W_e1[EMB:2 * EMB]).reshape(BB, N, HID)
    R = mm(act, W_e1[2 * EMB:]) + b_e1_ref[0]
    h = (P + R[:, None, :])[:, :, None, :] + Q[:, None, :, :]
    t = ln_relu(h, g_e_ref[0], be_e_ref[0])

    ii = jax.lax.broadcasted_iota(jnp.int32, (N, N), 0)
    jj = jax.lax.broadcasted_iota(jnp.int32, (N, N), 1)
    mask = jnp.where(ii == jj, 0.0, 1.0 / (N - 1))
    s = jnp.sum(t * mask[None, :, :, None], axis=2)

    W_n1 = W_n1_ref[0]
    Wn_g = W_n1[EMB + ACT:]
    W_eg = mm(W_e2_ref[0], Wn_g)
    c_eg = mm(b_e2_ref[0], Wn_g)
    U = mm(nf2, W_n1[0:EMB]).reshape(BB, N, HID)
    V = mm(act, W_n1[EMB:EMB + ACT]) + c_eg + b_n1_ref[0]
    G = mm(s.reshape(BB * N, HID), W_eg).reshape(BB, N, HID)
    h2 = U + G + V[:, None, :]
    t2 = ln_relu(h2, g_n_ref[0], be_n_ref[0])

    W_n2 = W_n2_ref[0]
    b_n2 = b_n2_ref[0]
    W_a = mm(W_n2, W_oa_ref[0])
    c_a = mm(b_n2, W_oa_ref[0]) + b_oa_ref[0]
    W_d = mm(W_n2, W_od_ref[0])
    c_d = mm(b_n2, W_od_ref[0]) + b_od_ref[0]
    agent_out_ref[0] = mm(t2[:, 0, :], W_a) + c_a
    obj = t2[:, 1:, :].reshape(BB * NOBJ, HID)
    obj_out_ref[0] = (mm(obj, W_d) + c_d).reshape(BB, NOBJ, DYN)


def kernel(agent_state, object_dyn_state, object_stat_state, action,
           W_ea, b_ea, W_eo, b_eo,
           W_e1, b_e1, g_e, be_e, W_e2, b_e2,
           W_n1, b_n1, g_n, be_n, W_n2, b_n2,
           W_oa, b_oa, W_od, b_od):
    grid = (NE, B // BB)

    b_ea, b_eo, b_e1, g_e, be_e, b_e2, b_n1, g_n, be_n, b_n2, b_oa, b_od = (
        x[:, None, :] for x in
        (b_ea, b_eo, b_e1, g_e, be_e, b_e2, b_n1, g_n, be_n, b_n2, b_oa, b_od))

    def eb(*blk):
        return pl.BlockSpec(blk, lambda e, b: (e, b) + (0,) * (len(blk) - 2))

    def ew(*blk):
        return pl.BlockSpec(blk, lambda e, b: (e,) + (0,) * (len(blk) - 1))

    in_specs = [
        eb(1, BB, AG),
        eb(1, BB, NOBJ, DYN),
        eb(1, BB, NOBJ, STAT),
        eb(1, BB, ACT),
        ew(1, AG, EMB), ew(1, 1, EMB),
        ew(1, DYN + STAT, EMB), ew(1, 1, EMB),
        ew(1, 2 * EMB + ACT, HID), ew(1, 1, HID),
        ew(1, 1, HID), ew(1, 1, HID),
        ew(1, HID, HID), ew(1, 1, HID),
        ew(1, EMB + HID + ACT, HID), ew(1, 1, HID),
        ew(1, 1, HID), ew(1, 1, HID),
        ew(1, HID, EMB), ew(1, 1, EMB),
        ew(1, EMB, AG), ew(1, 1, AG),
        ew(1, EMB, DYN), ew(1, 1, DYN),
    ]
    out_specs = (
        eb(1, BB, AG),
        eb(1, BB, NOBJ, DYN),
    )
    out_shapes = (
        jax.ShapeDtypeStruct((NE, B, AG), jnp.float32),
        jax.ShapeDtypeStruct((NE, B, NOBJ, DYN), jnp.float32),
    )
    return pl.pallas_call(
        _gnn_kernel,
        grid=grid,
        in_specs=in_specs,
        out_specs=out_specs,
        out_shape=out_shapes,
        compiler_params=pltpu.CompilerParams(
            dimension_semantics=("parallel", "parallel"),
        ),
    )(agent_state, object_dyn_state, object_stat_state, action,
      W_ea, b_ea, W_eo, b_eo,
      W_e1, b_e1, g_e, be_e, W_e2, b_e2,
      W_n1, b_n1, g_n, be_n, W_n2, b_n2,
      W_oa, b_oa, W_od, b_od)


# speedup vs baseline: 2.4319x; 2.4319x over previous
import jax
import jax.numpy as jnp
from jax.experimental import pallas as pl
from jax.experimental.pallas import tpu as pltpu

NE = 4
B = 256
NOBJ = 16
N = 17
AG = 8
DYN = 12
STAT = 4
EMB = 32
HID = 64
ACT = 8

BB = 64


def _gnn_kernel(agent_ref, dyn_ref, stat_ref, act_ref,
                W_ea_ref, b_ea_ref, W_eo_ref, b_eo_ref,
                W_e1_ref, b_e1_ref, g_e_ref, be_e_ref, W_e2_ref, b_e2_ref,
                W_n1_ref, b_n1_ref, g_n_ref, be_n_ref, W_n2_ref, b_n2_ref,
                W_oa_ref, b_oa_ref, W_od_ref, b_od_ref,
                agent_out_ref, obj_out_ref):
    f32 = jnp.float32

    def mm(x, w):
        return jnp.dot(x, w, preferred_element_type=f32)

    def ln_relu(h, g, bb):
        m = jnp.mean(h, axis=-1, keepdims=True)
        v = jnp.mean(jnp.square(h - m), axis=-1, keepdims=True)
        return jnp.maximum((h - m) * jax.lax.rsqrt(v + 1e-5) * g + bb, 0.0)

    J = jnp.full((HID, HID), 1.0 / HID, dtype=f32)

    def ln_relu2(h2d, g, bb):
        m = mm(h2d, J)
        q = mm(h2d * h2d, J)
        inv = jax.lax.rsqrt(q - m * m + 1e-5)
        return jnp.maximum((h2d - m) * inv * g + bb, 0.0)

    ag = agent_ref[0]
    act = act_ref[0]
    obj_inT = jnp.concatenate([dyn_ref[0], stat_ref[0]], axis=-1).transpose(1, 0, 2)

    agent_emb = mm(ag, W_ea_ref[0]) + b_ea_ref[0]
    obj_embT = mm(obj_inT.reshape(NOBJ * BB, DYN + STAT), W_eo_ref[0]) + b_eo_ref[0]
    nfT = jnp.concatenate([agent_emb, obj_embT], axis=0)

    W_e1 = W_e1_ref[0]
    Pd = mm(nfT, W_e1[0:EMB]).reshape(N, BB, HID)
    Qs = mm(nfT, W_e1[EMB:2 * EMB]).reshape(N, BB, HID)
    R = mm(act, W_e1[2 * EMB:]) + b_e1_ref[0]
    Pd = Pd + R[None, :, :]
    h = Qs[:, None, :, :] + Pd[None, :, :, :]
    g_e = g_e_ref[0]
    be_e = be_e_ref[0]
    t = ln_relu2(h.reshape(N * N * BB, HID), g_e, be_e)

    t_diag = ln_relu2(Qs.reshape(N * BB, HID) + Pd.reshape(N * BB, HID), g_e, be_e)
    s_full = jnp.sum(t.reshape(N, N * BB, HID), axis=0)
    s = (s_full - t_diag) * (1.0 / (N - 1))

    W_n1 = W_n1_ref[0]
    Wn_g = W_n1[EMB + ACT:]
    W_eg = mm(W_e2_ref[0], Wn_g)
    c_eg = mm(b_e2_ref[0], Wn_g)
    U = mm(nfT, W_n1[0:EMB])
    V = mm(act, W_n1[EMB:EMB + ACT]) + c_eg + b_n1_ref[0]
    G = mm(s, W_eg)
    h2 = (U + G).reshape(N, BB, HID) + V[None, :, :]
    t2 = ln_relu2(h2.reshape(N * BB, HID), g_n_ref[0], be_n_ref[0])
    t2 = t2.reshape(N, BB, HID)

    W_n2 = W_n2_ref[0]
    b_n2 = b_n2_ref[0]
    W_a = mm(W_n2, W_oa_ref[0])
    c_a = mm(b_n2, W_oa_ref[0]) + b_oa_ref[0]
    W_d = mm(W_n2, W_od_ref[0])
    c_d = mm(b_n2, W_od_ref[0]) + b_od_ref[0]
    agent_out_ref[0] = mm(t2[0], W_a) + c_a
    obj = t2[1:].reshape(NOBJ * BB, HID)
    obj_out_ref[0] = (mm(obj, W_d) + c_d).reshape(NOBJ, BB, DYN).transpose(1, 0, 2)


def kernel(agent_state, object_dyn_state, object_stat_state, action,
           W_ea, b_ea, W_eo, b_eo,
           W_e1, b_e1, g_e, be_e, W_e2, b_e2,
           W_n1, b_n1, g_n, be_n, W_n2, b_n2,
           W_oa, b_oa, W_od, b_od):
    grid = (NE, B // BB)

    b_ea, b_eo, b_e1, g_e, be_e, b_e2, b_n1, g_n, be_n, b_n2, b_oa, b_od = (
        x[:, None, :] for x in
        (b_ea, b_eo, b_e1, g_e, be_e, b_e2, b_n1, g_n, be_n, b_n2, b_oa, b_od))

    def eb(*blk):
        return pl.BlockSpec(blk, lambda e, b: (e, b) + (0,) * (len(blk) - 2))

    def ew(*blk):
        return pl.BlockSpec(blk, lambda e, b: (e,) + (0,) * (len(blk) - 1))

    in_specs = [
        eb(1, BB, AG),
        eb(1, BB, NOBJ, DYN),
        eb(1, BB, NOBJ, STAT),
        eb(1, BB, ACT),
        ew(1, AG, EMB), ew(1, 1, EMB),
        ew(1, DYN + STAT, EMB), ew(1, 1, EMB),
        ew(1, 2 * EMB + ACT, HID), ew(1, 1, HID),
        ew(1, 1, HID), ew(1, 1, HID),
        ew(1, HID, HID), ew(1, 1, HID),
        ew(1, EMB + HID + ACT, HID), ew(1, 1, HID),
        ew(1, 1, HID), ew(1, 1, HID),
        ew(1, HID, EMB), ew(1, 1, EMB),
        ew(1, EMB, AG), ew(1, 1, AG),
        ew(1, EMB, DYN), ew(1, 1, DYN),
    ]
    out_specs = (
        eb(1, BB, AG),
        eb(1, BB, NOBJ, DYN),
    )
    out_shapes = (
        jax.ShapeDtypeStruct((NE, B, AG), jnp.float32),
        jax.ShapeDtypeStruct((NE, B, NOBJ, DYN), jnp.float32),
    )
    return pl.pallas_call(
        _gnn_kernel,
        grid=grid,
        in_specs=in_specs,
        out_specs=out_specs,
        out_shape=out_shapes,
        compiler_params=pltpu.CompilerParams(
            dimension_semantics=("parallel", "parallel"),
        ),
    )(agent_state, object_dyn_state, object_stat_state, action,
      W_ea, b_ea, W_eo, b_eo,
      W_e1, b_e1, g_e, be_e, W_e2, b_e2,
      W_n1, b_n1, g_n, be_n, W_n2, b_n2,
      W_oa, b_oa, W_od, b_od)
